# Initial kernel scaffold; baseline (speedup 1.0000x reference)
import functools

import jax
import jax.numpy as jnp
from jax import lax
from jax.experimental import pallas as pl
from jax.experimental.pallas import tpu as pltpu
from jax.experimental.pallas import tpu_sc as plsc

N_CTX = 100000
N_TACTICS = 1000
DIM = 128
BATCH = 4096
BAG = 50

NC = 2
NS = 16
NW = NC * NS
BPW = BATCH // NW
K = 4
CHUNK_ROWS = K * BAG
NCHUNK = BPW // K
IDX_ROW = 100
SUB = CHUNK_ROWS // IDX_ROW
NCOL = DIM // 16


def _sc_kernel(ctx2d, t_ids, c_in_w, t_out_w):
    mesh = plsc.VectorSubcoreMesh(core_axis_name="c", subcore_axis_name="s")

    @functools.partial(
        pl.kernel,
        mesh=mesh,
        out_type=jax.ShapeDtypeStruct((BATCH,), jnp.float32),
        scratch_types=[
            pltpu.VMEM((BPW * BAG // IDX_ROW, IDX_ROW), jnp.int32),
            pltpu.VMEM((BPW,), jnp.int32),
            pltpu.VMEM((BPW, DIM), jnp.float32),
            pltpu.VMEM((CHUNK_ROWS, DIM), jnp.float32),
            pltpu.VMEM((CHUNK_ROWS, DIM), jnp.float32),
            pltpu.VMEM((BPW,), jnp.float32),
            pltpu.SemaphoreType.DMA,
            pltpu.SemaphoreType.DMA,
            pltpu.SemaphoreType.DMA,
        ],
    )
    def body(ctx_hbm, tids_hbm, cin_hbm, tout_hbm, out_hbm,
             idx_v, tidx_v, tbuf, buf0, buf1, scv, sem0, sem1, semt):
        wid = lax.axis_index("s") * NC + lax.axis_index("c")
        idx_rows = BPW * BAG // IDX_ROW

        pltpu.sync_copy(ctx_hbm.at[pl.ds(wid * idx_rows, idx_rows)], idx_v)
        pltpu.sync_copy(tids_hbm.at[pl.ds(wid * BPW, BPW)], tidx_v)
        tcp = pltpu.async_copy(tout_hbm.at[tidx_v], tbuf, semt)

        bufs = (buf0, buf1)
        sems = (sem0, sem1)

        def fire(c):
            b = c % 2
            cps = []
            for i in range(SUB):
                cps.append(pltpu.async_copy(
                    cin_hbm.at[idx_v.at[c * SUB + i]],
                    bufs[b].at[pl.ds(i * IDX_ROW, IDX_ROW)],
                    sems[b]))
            return cps

        pend = fire(0)
        tcp.wait()

        lanes = lax.iota(jnp.int32, 16)
        score_vec = jnp.zeros((16,), jnp.float32)

        for c in range(NCHUNK):
            buf = bufs[c % 2]
            for cp in pend:
                cp.wait()
            if c + 1 < NCHUNK:
                pend = fire(c + 1)

            def bag_body(m, sv):
                row0 = m * BAG
                accs = tuple(buf[row0, pl.ds(cc * 16, 16)] for cc in range(NCOL))

                def jbody(j, a):
                    return tuple(a[cc] + buf[row0 + j, pl.ds(cc * 16, 16)]
                                 for cc in range(NCOL))

                accs = lax.fori_loop(1, BAG, jbody, accs)
                tb = c * K + m
                tot = accs[0] * tbuf[tb, pl.ds(0, 16)]
                for cc in range(1, NCOL):
                    tot = tot + accs[cc] * tbuf[tb, pl.ds(cc * 16, 16)]
                s = jnp.sum(tot)
                return jnp.where(lanes == (tb % 16), s, sv)

            score_vec = lax.fori_loop(0, K, bag_body, score_vec)

            if c % 4 == 3:
                scv[pl.ds((c // 4) * 16, 16)] = score_vec * (1.0 / BAG)

        pltpu.sync_copy(scv, out_hbm.at[pl.ds(wid * BPW, BPW)])

    return body(ctx2d, t_ids, c_in_w, t_out_w)


def kernel(ctx_ids, t_ids, c_in_w, t_out_w):
    ctx2d = ctx_ids.reshape(BATCH * BAG // IDX_ROW, IDX_ROW)
    return _sc_kernel(ctx2d, t_ids, c_in_w, t_out_w)


# speedup vs baseline: 11.1522x; 11.1522x over previous
import functools

import jax
import jax.numpy as jnp
from jax import lax
from jax.experimental import pallas as pl
from jax.experimental.pallas import tpu as pltpu
from jax.experimental.pallas import tpu_sc as plsc

N_CTX = 100000
N_TACTICS = 1000
DIM = 128
BATCH = 4096
BAG = 50

NC = 2
NS = 16
NW = NC * NS
BPW = BATCH // NW
K = 4
CHUNK_ROWS = K * BAG
NCHUNK = BPW // K
IDX_ROW = 100
SUB = CHUNK_ROWS // IDX_ROW
NCOL = DIM // 16


def _sc_kernel(ctx2d, t_ids, c_in_w, t_out_w):
    mesh = plsc.VectorSubcoreMesh(core_axis_name="c", subcore_axis_name="s")

    @functools.partial(
        pl.kernel,
        mesh=mesh,
        compiler_params=pltpu.CompilerParams(needs_layout_passes=False),
        out_type=jax.ShapeDtypeStruct((BATCH,), jnp.float32),
        scratch_types=[
            pltpu.VMEM((BPW * BAG // IDX_ROW, IDX_ROW), jnp.int32),
            pltpu.VMEM((BPW,), jnp.int32),
            pltpu.VMEM((BPW, DIM), jnp.float32),
            pltpu.VMEM((CHUNK_ROWS, DIM), jnp.float32),
            pltpu.VMEM((CHUNK_ROWS, DIM), jnp.float32),
            pltpu.VMEM((BPW,), jnp.float32),
            pltpu.VMEM((256,), jnp.float32),
            pltpu.SemaphoreType.DMA,
            pltpu.SemaphoreType.DMA,
            pltpu.SemaphoreType.DMA,
        ],
    )
    def body(ctx_hbm, tids_hbm, cin_hbm, tout_hbm, out_hbm,
             idx_v, tidx_v, tbuf, buf0, buf1, scv, part, sem0, sem1, semt):
        wid = lax.axis_index("s") * NC + lax.axis_index("c")
        idx_rows = BPW * BAG // IDX_ROW

        pltpu.sync_copy(ctx_hbm.at[pl.ds(wid * idx_rows, idx_rows)], idx_v)
        pltpu.sync_copy(tids_hbm.at[pl.ds(wid * BPW, BPW)], tidx_v)
        tcp = pltpu.async_copy(tout_hbm.at[tidx_v], tbuf, semt)

        bufs = (buf0, buf1)
        sems = (sem0, sem1)

        def fire(c):
            b = c % 2
            cps = []
            for i in range(SUB):
                cps.append(pltpu.async_copy(
                    cin_hbm.at[idx_v.at[c * SUB + i]],
                    bufs[b].at[pl.ds(i * IDX_ROW, IDX_ROW)],
                    sems[b]))
            return cps

        pend = fire(0)
        tcp.wait()

        lanes16 = lax.iota(jnp.int32, 16) * 16

        for c in range(NCHUNK):
            buf = bufs[c % 2]
            for cp in pend:
                cp.wait()
            if c + 1 < NCHUNK:
                pend = fire(c + 1)

            def bag_body(m, carry):
                row0 = m * BAG
                accs = tuple(buf[row0, pl.ds(cc * 16, 16)] for cc in range(NCOL))

                def jbody(j, a):
                    return tuple(a[cc] + buf[row0 + j, pl.ds(cc * 16, 16)]
                                 for cc in range(NCOL))

                accs = lax.fori_loop(1, BAG, jbody, accs)
                tb = c * K + m
                tot = accs[0] * tbuf[tb, pl.ds(0, 16)]
                for cc in range(1, NCOL):
                    tot = tot + accs[cc] * tbuf[tb, pl.ds(cc * 16, 16)]
                plsc.store_scatter(part, [lanes16 + (tb % 16)], tot)
                return carry

            lax.fori_loop(0, K, bag_body, jnp.int32(0))

            if c % 4 == 3:
                s = part[pl.ds(0, 16)]
                for r in range(1, 16):
                    s = s + part[pl.ds(r * 16, 16)]
                scv[pl.ds((c // 4) * 16, 16)] = s * (1.0 / BAG)

        pltpu.sync_copy(scv, out_hbm.at[pl.ds(wid * BPW, BPW)])

    return body(ctx2d, t_ids, c_in_w, t_out_w)


def kernel(ctx_ids, t_ids, c_in_w, t_out_w):
    ctx2d = ctx_ids.reshape(BATCH * BAG // IDX_ROW, IDX_ROW)
    return _sc_kernel(ctx2d, t_ids, c_in_w, t_out_w)


# speedup vs baseline: 13.9591x; 1.2517x over previous
import functools

import jax
import jax.numpy as jnp
from jax import lax
from jax.experimental import pallas as pl
from jax.experimental.pallas import tpu as pltpu
from jax.experimental.pallas import tpu_sc as plsc

N_CTX = 100000
N_TACTICS = 1000
DIM = 128
BATCH = 4096
BAG = 50

NC = 2
NS = 16
NW = NC * NS
BPW = BATCH // NW
K = 2
CHUNK_ROWS = K * BAG
NCHUNK = BPW // K
RING = 6
NCOL = DIM // 16


def _sc_kernel(ctx2d, t_ids, c_in_w, t_out_w):
    mesh = plsc.VectorSubcoreMesh(core_axis_name="c", subcore_axis_name="s")

    @functools.partial(
        pl.kernel,
        mesh=mesh,
        compiler_params=pltpu.CompilerParams(needs_layout_passes=False),
        out_type=jax.ShapeDtypeStruct((BATCH,), jnp.float32),
        scratch_types=[
            pltpu.VMEM((NCHUNK, CHUNK_ROWS), jnp.int32),
            pltpu.VMEM((BPW,), jnp.int32),
            pltpu.VMEM((BPW, DIM), jnp.float32),
        ]
        + [pltpu.VMEM((CHUNK_ROWS, DIM), jnp.float32)] * RING
        + [
            pltpu.VMEM((BPW,), jnp.float32),
            pltpu.VMEM((256,), jnp.float32),
        ]
        + [pltpu.SemaphoreType.DMA] * RING
        + [pltpu.SemaphoreType.DMA],
    )
    def body(ctx_hbm, tids_hbm, cin_hbm, tout_hbm, out_hbm,
             idx_v, tidx_v, tbuf, *rest):
        bufs = rest[:RING]
        scv, part = rest[RING], rest[RING + 1]
        sems = rest[RING + 2:]
        semt = sems[RING]
        wid = lax.axis_index("s") * NC + lax.axis_index("c")

        pltpu.sync_copy(ctx_hbm.at[pl.ds(wid * NCHUNK, NCHUNK)], idx_v)
        pltpu.sync_copy(tids_hbm.at[pl.ds(wid * BPW, BPW)], tidx_v)
        tcp = pltpu.async_copy(tout_hbm.at[tidx_v], tbuf, semt)

        def fire(c):
            s = c % RING
            return pltpu.async_copy(cin_hbm.at[idx_v.at[c]], bufs[s], sems[s])

        pend = [fire(c) for c in range(RING)]
        tcp.wait()

        lanes16 = lax.iota(jnp.int32, 16) * 16

        for c in range(NCHUNK):
            s = c % RING
            buf = bufs[s]
            pend[s].wait()

            def bag_body(m, carry):
                row0 = m * BAG
                accs = tuple(buf[row0, pl.ds(cc * 16, 16)]
                             for cc in range(NCOL))

                def jbody(j, a):
                    return tuple(a[cc] + buf[row0 + j, pl.ds(cc * 16, 16)]
                                 for cc in range(NCOL))

                accs = lax.fori_loop(1, BAG, jbody, accs)
                tb = c * K + m
                tot = accs[0] * tbuf[tb, pl.ds(0, 16)]
                for cc in range(1, NCOL):
                    tot = tot + accs[cc] * tbuf[tb, pl.ds(cc * 16, 16)]
                plsc.store_scatter(part, [lanes16 + (tb % 16)], tot)
                return carry

            lax.fori_loop(0, K, bag_body, jnp.int32(0))

            if c + RING < NCHUNK:
                pend[s] = fire(c + RING)

            if c % 8 == 7:
                acc = part[pl.ds(0, 16)]
                for r in range(1, 16):
                    acc = acc + part[pl.ds(r * 16, 16)]
                scv[pl.ds((c // 8) * 16, 16)] = acc * (1.0 / BAG)

        pltpu.sync_copy(scv, out_hbm.at[pl.ds(wid * BPW, BPW)])

    return body(ctx2d, t_ids, c_in_w, t_out_w)


def kernel(ctx_ids, t_ids, c_in_w, t_out_w):
    ctx2d = ctx_ids.reshape(BATCH * BAG // CHUNK_ROWS, CHUNK_ROWS)
    return _sc_kernel(ctx2d, t_ids, c_in_w, t_out_w)
